# baseline (speedup 1.0000x reference)
import jax
import jax.numpy as jnp
from jax.experimental import pallas as pl

BATCH = 32
MAX_SEQ_LEN = 4096
KV_HEAD_DIM = 128
PREFILL_LEN = 2048

BATCH_BLOCK = 4
N_BLOCKS = BATCH // BATCH_BLOCK


def _body(x_ref, out_ref):
    out_ref[:, :PREFILL_LEN, :] = x_ref[...]
    out_ref[:, PREFILL_LEN:, :] = jnp.zeros(
        (BATCH_BLOCK, MAX_SEQ_LEN - PREFILL_LEN, KV_HEAD_DIM), jnp.float32
    )


def _prefill_one(x):
    in_spec = pl.BlockSpec(
        (BATCH_BLOCK, PREFILL_LEN, KV_HEAD_DIM),
        lambda j: (j, 0, 0),
    )
    out_spec = pl.BlockSpec(
        (BATCH_BLOCK, MAX_SEQ_LEN, KV_HEAD_DIM),
        lambda j: (j, 0, 0),
    )
    return pl.pallas_call(
        _body,
        grid=(N_BLOCKS,),
        in_specs=[in_spec],
        out_specs=out_spec,
        out_shape=jax.ShapeDtypeStruct((BATCH, MAX_SEQ_LEN, KV_HEAD_DIM), jnp.float32),
    )(x)


def kernel(kx, vx):
    return (_prefill_one(kx), _prefill_one(vx))


# speedup vs baseline: 1.0551x; 1.0551x over previous
import jax
import jax.numpy as jnp
from jax.experimental import pallas as pl

BATCH = 32
MAX_SEQ_LEN = 4096
KV_HEAD_DIM = 128
PREFILL_LEN = 2048

BATCH_BLOCK = 8
N_BLOCKS = BATCH // BATCH_BLOCK


def _body(x_ref, out_ref):
    out_ref[:, :PREFILL_LEN, :] = x_ref[...]
    out_ref[:, PREFILL_LEN:, :] = jnp.zeros(
        (BATCH_BLOCK, MAX_SEQ_LEN - PREFILL_LEN, KV_HEAD_DIM), jnp.float32
    )


def _prefill_one(x):
    in_spec = pl.BlockSpec(
        (BATCH_BLOCK, PREFILL_LEN, KV_HEAD_DIM),
        lambda j: (j, 0, 0),
    )
    out_spec = pl.BlockSpec(
        (BATCH_BLOCK, MAX_SEQ_LEN, KV_HEAD_DIM),
        lambda j: (j, 0, 0),
    )
    return pl.pallas_call(
        _body,
        grid=(N_BLOCKS,),
        in_specs=[in_spec],
        out_specs=out_spec,
        out_shape=jax.ShapeDtypeStruct((BATCH, MAX_SEQ_LEN, KV_HEAD_DIM), jnp.float32),
    )(x)


def kernel(kx, vx):
    return (_prefill_one(kx), _prefill_one(vx))
